# baseline (speedup 1.0000x reference)
import functools

import jax
import jax.numpy as jnp
import numpy as np
from jax import lax
from jax.experimental import pallas as pl
from jax.experimental.pallas import tpu as pltpu
from jax.experimental.pallas import tpu_sc as plsc

_N = 100000
_NPAD = 100352
_ROWS = 784
_K = 4096
_NROI = 128
_BIG = np.int32(2**30)

_PC_X0 = np.float32(-75.2)
_VOX = np.float32(0.1)
_STRIDE = np.float32(8.0)
_RADIUS = np.float32(1.6)
_H = 188
_W = 188
_C = 256


def _fps_body(xs_ref, ys_ref, zs_ref, rois_ref, kp_ref, bd_ref, bt_ref):
    i32 = jnp.int32
    row_i = lax.broadcasted_iota(i32, (_ROWS, 128), 0)
    col_i = lax.broadcasted_iota(i32, (_ROWS, 128), 1)
    lin = row_i * 128 + col_i
    valid = lin < _N

    xs = xs_ref[...]
    ys = ys_ref[...]
    zs = zs_ref[...]

    bd_ref[...] = jnp.full((_ROWS, 128), jnp.inf, jnp.float32)
    bt_ref[...] = jnp.zeros((_ROWS, 128), jnp.float32)

    def roi_body(j, _):
        r = rois_ref[pl.ds(j, 1), :]
        cx = r[0, 0]
        cy = r[0, 1]
        cz = r[0, 2]
        hx = r[0, 3] / 2.0
        hy = r[0, 4] / 2.0
        hz = r[0, 5] / 2.0
        thr = jnp.sqrt((hx * hx + hy * hy) + hz * hz) + _RADIUS
        dx = xs - cx
        dy = ys - cy
        dz = zs - cz
        dist = jnp.sqrt((dx * dx + dy * dy) + dz * dz)
        bd = bd_ref[...]
        pred = dist < bd
        bt_ref[...] = jnp.where(pred, thr, bt_ref[...])
        bd_ref[...] = jnp.where(pred, dist, bd)
        return 0

    lax.fori_loop(0, _NROI, roi_body, 0)

    mask = (bd_ref[...] < bt_ref[...]) & valid
    first = jnp.min(jnp.where(mask, lin, _BIG))
    first = jnp.where(first == _BIG, 0, first).astype(i32)

    bd_ref[...] = jnp.where(mask, jnp.float32(1e10), jnp.float32(-1.0))

    lane = lax.broadcasted_iota(i32, (1, 128), 1)
    oh0 = (lane == 0).astype(jnp.float32)
    oh1 = (lane == 1).astype(jnp.float32)
    oh2 = (lane == 2).astype(jnp.float32)

    def extract(idx):
        r = idx // 128
        c = idx % 128
        m = (lane == c).astype(jnp.float32)
        px = jnp.sum(xs_ref[pl.ds(r, 1), :] * m)
        py = jnp.sum(ys_ref[pl.ds(r, 1), :] * m)
        pz = jnp.sum(zs_ref[pl.ds(r, 1), :] * m)
        return px, py, pz

    def body(i, last):
        px, py, pz = extract(last)
        kp_ref[pl.ds(i - 1, 1), :] = px * oh0 + py * oh1 + pz * oh2
        dx = xs - px
        dy = ys - py
        dz = zs - pz
        d = (dx * dx + dy * dy) + dz * dz
        md = jnp.minimum(bd_ref[...], d)
        bd_ref[...] = md
        m = jnp.max(md)
        nxt = jnp.min(jnp.where(md == m, lin, _BIG)).astype(i32)
        return nxt

    last = lax.fori_loop(1, _K, body, first)
    px, py, pz = extract(last)
    kp_ref[pl.ds(_K - 1, 1), :] = px * oh0 + py * oh1 + pz * oh2


def _stage1(xs, ys, zs, rois8):
    return pl.pallas_call(
        _fps_body,
        out_shape=jax.ShapeDtypeStruct((_K, 128), jnp.float32),
        scratch_shapes=[
            pltpu.VMEM((_ROWS, 128), jnp.float32),
            pltpu.VMEM((_ROWS, 128), jnp.float32),
        ],
    )(xs, ys, zs, rois8)


def _sc_body(kpx_hbm, kpy_hbm, table_hbm, out4_hbm, w4_hbm,
             kpx_v, kpy_v, ia_v, ib_v, ic_v, id_v,
             wa_v, wb_v, wc_v, wd_v, buf0, buf1, sem0, sem1):
    i32 = jnp.int32
    f32 = jnp.float32
    wid = lax.axis_index("s") * 2 + lax.axis_index("c")
    base = wid * 128
    pltpu.sync_copy(kpx_hbm.at[pl.ds(base, 128)], kpx_v)
    pltpu.sync_copy(kpy_hbm.at[pl.ds(base, 128)], kpy_v)

    def _floor(v):
        t = v.astype(i32)
        return t - jnp.where(t.astype(f32) > v, 1, 0)

    for c in range(8):
        sl = pl.ds(c * 16, 16)
        x = kpx_v[sl]
        y = kpy_v[sl]
        xi = (x - _PC_X0) / _VOX / _STRIDE
        yi = (y - _PC_X0) / _VOX / _STRIDE
        x0i = _floor(xi)
        y0i = _floor(yi)
        x0f = x0i.astype(f32)
        y0f = y0i.astype(f32)
        x1f = x0f + 1.0
        y1f = y0f + 1.0
        x0 = jnp.clip(x0i, 0, _W - 1)
        x1 = jnp.clip(x0i + 1, 0, _W - 1)
        y0 = jnp.clip(y0i, 0, _H - 1)
        y1 = jnp.clip(y0i + 1, 0, _H - 1)
        ia_v[sl] = y0 * _W + x0
        ib_v[sl] = y1 * _W + x0
        ic_v[sl] = y0 * _W + x1
        id_v[sl] = y1 * _W + x1
        wa_v[sl] = (x1f - xi) * (y1f - yi)
        wb_v[sl] = (x1f - xi) * (yi - y0f)
        wc_v[sl] = (xi - x0f) * (y1f - yi)
        wd_v[sl] = (xi - x0f) * (yi - y0f)

    pltpu.sync_copy(wa_v, w4_hbm.at[0, pl.ds(base, 128)])
    pltpu.sync_copy(wb_v, w4_hbm.at[1, pl.ds(base, 128)])
    pltpu.sync_copy(wc_v, w4_hbm.at[2, pl.ds(base, 128)])
    pltpu.sync_copy(wd_v, w4_hbm.at[3, pl.ds(base, 128)])

    idxs = (ia_v, ib_v, ic_v, id_v)
    bufs = (buf0, buf1)
    sems = (sem0, sem1)
    pending = pltpu.async_copy(table_hbm.at[ia_v], buf0, sem0)
    for c in range(4):
        nxt = None
        if c < 3:
            nxt = pltpu.async_copy(
                table_hbm.at[idxs[c + 1]], bufs[(c + 1) % 2], sems[(c + 1) % 2])
        pending.wait()
        pltpu.sync_copy(bufs[c % 2], out4_hbm.at[c, pl.ds(base, 128)])
        pending = nxt


def _stage2(kpx, kpy, table):
    mesh = plsc.VectorSubcoreMesh(core_axis_name="c", subcore_axis_name="s")
    f = functools.partial(
        pl.kernel,
        mesh=mesh,
        out_type=[
            jax.ShapeDtypeStruct((4, _K, _C), jnp.float32),
            jax.ShapeDtypeStruct((4, _K), jnp.float32),
        ],
        scratch_types=[
            pltpu.VMEM((128,), jnp.float32),
            pltpu.VMEM((128,), jnp.float32),
            pltpu.VMEM((128,), jnp.int32),
            pltpu.VMEM((128,), jnp.int32),
            pltpu.VMEM((128,), jnp.int32),
            pltpu.VMEM((128,), jnp.int32),
            pltpu.VMEM((128,), jnp.float32),
            pltpu.VMEM((128,), jnp.float32),
            pltpu.VMEM((128,), jnp.float32),
            pltpu.VMEM((128,), jnp.float32),
            pltpu.VMEM((128, _C), jnp.float32),
            pltpu.VMEM((128, _C), jnp.float32),
            pltpu.SemaphoreType.DMA,
            pltpu.SemaphoreType.DMA,
        ],
    )(_sc_body)
    return f(kpx, kpy, table)


def _comb_body(x_ref, w_ref, o_ref):
    a = x_ref[0] * w_ref[0]
    b = x_ref[1] * w_ref[1]
    c = x_ref[2] * w_ref[2]
    d = x_ref[3] * w_ref[3]
    o_ref[...] = ((a + b) + c) + d


def _combine(out4, w4):
    r = 1024
    return pl.pallas_call(
        _comb_body,
        grid=(_K // r,),
        in_specs=[
            pl.BlockSpec((4, r, _C), lambda i: (0, i, 0)),
            pl.BlockSpec((4, r, 1), lambda i: (0, i, 0)),
        ],
        out_specs=pl.BlockSpec((r, _C), lambda i: (i, 0)),
        out_shape=jax.ShapeDtypeStruct((_K, _C), jnp.float32),
    )(out4, w4.reshape(4, _K, 1))


def kernel(points, rois, bev_features):
    pad = jnp.zeros((_NPAD - _N, 3), jnp.float32)
    pp = jnp.concatenate([points, pad], axis=0)
    xs = pp[:, 0].reshape(_ROWS, 128)
    ys = pp[:, 1].reshape(_ROWS, 128)
    zs = pp[:, 2].reshape(_ROWS, 128)
    rois8 = jnp.pad(rois, ((0, 0), (0, 1)))
    kp = _stage1(xs, ys, zs, rois8)
    table = jnp.transpose(bev_features, (1, 2, 0)).reshape(_H * _W, _C)
    out4, w4 = _stage2(kp[:, 0], kp[:, 1], table)
    bev_feats = _combine(out4, w4)
    return jnp.concatenate([kp[:, :3], bev_feats], axis=1)
